# Initial kernel scaffold; baseline (speedup 1.0000x reference)
import jax
import jax.numpy as jnp
from jax.experimental import pallas as pl


def kernel(q, k, v):
    raise NotImplementedError("write your pallas kernel here")


# speedup vs baseline: 5.5243x; 5.5243x over previous
import functools

import jax
import jax.numpy as jnp
from jax.experimental import pallas as pl
from jax.experimental.pallas import tpu as pltpu

_TOPK = 128
_BQ = 128
_NEG = -1e9


def _float_keys(s):
    si = jax.lax.bitcast_convert_type(s, jnp.int32)
    return jnp.where(si < 0, si ^ jnp.int32(0x7FFFFFFF), si)


def _attn_body(q_ref, k_ref, v_ref, o_ref):
    qb = pl.program_id(1)
    q = q_ref[0]
    k = k_ref[0]
    v = v_ref[0]
    bq, d = q.shape
    s_len = k.shape[0]
    scale = jnp.float32(1.0) / jnp.sqrt(jnp.float32(d))

    s = jax.lax.dot_general(
        q, k, (((1,), (1,)), ((), ())),
        preferred_element_type=jnp.float32,
        precision=jax.lax.Precision.DEFAULT) * scale

    row = qb * bq + jax.lax.broadcasted_iota(jnp.int32, (bq, s_len), 0)
    col = jax.lax.broadcasted_iota(jnp.int32, (bq, s_len), 1)
    s = jnp.where(col <= row, s, jnp.float32(_NEG))

    keys = _float_keys(s)
    lo = jnp.min(keys, axis=-1, keepdims=True)
    hi = jnp.max(keys, axis=-1, keepdims=True) + 1

    def bisect(_, carry):
        lo, hi = carry
        mid = (lo >> 1) + (hi >> 1) + (lo & hi & 1)
        cnt = jnp.sum((keys >= mid).astype(jnp.int32), axis=-1, keepdims=True)
        ge = cnt >= _TOPK
        return jnp.where(ge, mid, lo), jnp.where(ge, hi, mid)

    lo, hi = jax.lax.fori_loop(0, 32, bisect, (lo, hi))

    m = jnp.max(s, axis=-1, keepdims=True)
    p = jnp.where(keys >= lo, jnp.exp(s - m), jnp.float32(0.0))
    den = jnp.sum(p, axis=-1, keepdims=True)
    o = jax.lax.dot_general(
        p, v, (((1,), (0,)), ((), ())),
        preferred_element_type=jnp.float32,
        precision=jax.lax.Precision.DEFAULT)
    o_ref[0] = o / den


def _build_call(bh, s_len, d, interpret=False):
    grid = (bh, s_len // _BQ)
    return pl.pallas_call(
        _attn_body,
        grid=grid,
        in_specs=[
            pl.BlockSpec((1, _BQ, d), lambda b, i: (b, i, 0)),
            pl.BlockSpec((1, s_len, d), lambda b, i: (b, 0, 0)),
            pl.BlockSpec((1, s_len, d), lambda b, i: (b, 0, 0)),
        ],
        out_specs=pl.BlockSpec((1, _BQ, d), lambda b, i: (b, i, 0)),
        out_shape=jax.ShapeDtypeStruct((bh, s_len, d), jnp.float32),
        compiler_params=pltpu.CompilerParams(
            dimension_semantics=("parallel", "arbitrary")),
        interpret=interpret,
    )


@jax.jit
def kernel(q, k, v):
    b, h, s_len, d = q.shape
    qf = q.reshape(b * h, s_len, d)
    kf = k.reshape(b * h, s_len, d)
    vf = v.reshape(b * h, s_len, d)
    out = _build_call(b * h, s_len, d)(qf, kf, vf)
    return out.reshape(b, h, s_len, d)


# speedup vs baseline: 9.4625x; 1.7129x over previous
import functools

import jax
import jax.numpy as jnp
from jax.experimental import pallas as pl
from jax.experimental.pallas import tpu as pltpu

_TOPK = 128
_BQ = 128
_NEG = -1e9


def _float_keys(s):
    si = jax.lax.bitcast_convert_type(s, jnp.int32)
    return jnp.where(si < 0, si ^ jnp.int32(0x7FFFFFFF), si)


def _attn_body(q_ref, k_ref, v_ref, o_ref):
    qb = pl.program_id(1)
    q = q_ref[0]
    k = k_ref[0]
    v = v_ref[0]
    bq, d = q.shape
    s_len = k.shape[0]
    scale = jnp.float32(1.0) / jnp.sqrt(jnp.float32(d))

    s = jax.lax.dot_general(
        q, k, (((1,), (1,)), ((), ())),
        preferred_element_type=jnp.float32,
        precision=jax.lax.Precision.DEFAULT) * scale

    row = qb * bq + jax.lax.broadcasted_iota(jnp.int32, (bq, s_len), 0)
    col = jax.lax.broadcasted_iota(jnp.int32, (bq, s_len), 1)
    s = jnp.where(col <= row, s, jnp.float32(_NEG))

    m = jnp.max(s, axis=-1, keepdims=True)
    lo = m - jnp.float32(25.0)
    hi = m

    def bisect(_, carry):
        lo, hi = carry
        mid = jnp.float32(0.5) * (lo + hi)
        cnt = jnp.sum(jnp.where(s >= mid, jnp.float32(1.0),
                                jnp.float32(0.0)), axis=-1, keepdims=True)
        ge = cnt >= _TOPK
        return jnp.where(ge, mid, lo), jnp.where(ge, hi, mid)

    lo, hi = jax.lax.fori_loop(0, 20, bisect, (lo, hi))

    p = jnp.where(s >= lo, jnp.exp(s - m), jnp.float32(0.0))
    den = jnp.sum(p, axis=-1, keepdims=True)
    o = jax.lax.dot_general(
        p, v, (((1,), (0,)), ((), ())),
        preferred_element_type=jnp.float32,
        precision=jax.lax.Precision.DEFAULT)
    o_ref[0] = o / den


def _build_call(bh, s_len, d, interpret=False):
    grid = (bh, s_len // _BQ)
    return pl.pallas_call(
        _attn_body,
        grid=grid,
        in_specs=[
            pl.BlockSpec((1, _BQ, d), lambda b, i: (b, i, 0)),
            pl.BlockSpec((1, s_len, d), lambda b, i: (b, 0, 0)),
            pl.BlockSpec((1, s_len, d), lambda b, i: (b, 0, 0)),
        ],
        out_specs=pl.BlockSpec((1, _BQ, d), lambda b, i: (b, i, 0)),
        out_shape=jax.ShapeDtypeStruct((bh, s_len, d), jnp.float32),
        compiler_params=pltpu.CompilerParams(
            dimension_semantics=("parallel", "arbitrary")),
        interpret=interpret,
    )


@jax.jit
def kernel(q, k, v):
    b, h, s_len, d = q.shape
    qf = q.reshape(b * h, s_len, d)
    kf = k.reshape(b * h, s_len, d)
    vf = v.reshape(b * h, s_len, d)
    out = _build_call(b * h, s_len, d)(qf, kf, vf)
    return out.reshape(b, h, s_len, d)
